# baseline (speedup 1.0000x reference)
import functools

import jax
import jax.numpy as jnp
import numpy as np
from jax import lax
from jax.experimental import pallas as pl
from jax.experimental.pallas import tpu as pltpu
from jax.experimental.pallas import tpu_sc as plsc

B = 16384
F = 26
D = 16
VOCAB_PER_FIELD = 100000
N = B * F
NC, NS = 2, 16
NW = NC * NS
PER_W = N // NW
CHUNK = 1664
ROWS = CHUNK // F
CHUNKS = PER_W // CHUNK
D_IN = F * D
WIDE = 512
H1, H2 = 256, 128
BB = 1024


V = 2600000
NGRP = V // 8
VC = 2048
VCHUNKS = -(-V // VC)
W_CHUNKS = -(-VCHUNKS // NW)
V_TAIL = (V // 128) * 128
V_LAST = V - V_TAIL


def _sc_untile(embT):
    mesh = plsc.VectorSubcoreMesh(core_axis_name="c", subcore_axis_name="s")

    @functools.partial(
        pl.kernel,
        out_type=jax.ShapeDtypeStruct((NGRP, 128), jnp.float32),
        mesh=mesh,
        scratch_types=(
            pltpu.VMEM((D, VC), jnp.float32),
            pltpu.VMEM((VC // 8, 128), jnp.float32),
        ),
        compiler_params=pltpu.CompilerParams(use_tc_tiling_on_sc=True,
                                             needs_layout_passes=False),
    )
    def k(embT_hbm, out_hbm, tbuf, obuf):
        wid = lax.axis_index("s") * NC + lax.axis_index("c")
        lane_iota = lax.iota(jnp.int32, 16)

        def transpose_block(nv):
            def tr(i, cc):
                vv = lane_iota + i * 16
                orow = lax.shift_right_logical(vv, 3)
                ocol0 = lax.bitwise_and(vv, 7) * 16
                for d in range(D):
                    vals = tbuf[d, pl.ds(pl.multiple_of(i * 16, 16), 16)]
                    plsc.store_scatter(obuf, [orow, ocol0 + d], vals)
                return cc

            lax.fori_loop(0, nv // 16, tr, 0)

        def chunk(t, carry):
            c = wid + t * NW
            v0 = jnp.minimum(c * VC, V_TAIL - VC)
            v0 = pl.multiple_of(v0, 128)
            pltpu.sync_copy(embT_hbm.at[:, pl.ds(v0, VC)], tbuf)
            transpose_block(VC)
            r0 = pl.multiple_of(lax.shift_right_logical(v0, 3), 8)
            pltpu.sync_copy(obuf, out_hbm.at[pl.ds(r0, VC // 8)])
            return carry

        lax.fori_loop(0, W_CHUNKS, chunk, 0)


    return k(embT)


def _sc_gather(x_flat, emb_table, lin16, off_flat, tail16):
    mesh = plsc.VectorSubcoreMesh(core_axis_name="c", subcore_axis_name="s")

    @functools.partial(
        pl.kernel,
        out_type=jax.ShapeDtypeStruct((B, WIDE), jnp.float32),
        mesh=mesh,
        scratch_types=(
            pltpu.VMEM((CHUNK,), jnp.int32),
            pltpu.VMEM((CHUNK,), jnp.int32),
            pltpu.VMEM((CHUNK,), jnp.int32),
            pltpu.VMEM((CHUNK, D), jnp.float32),
            pltpu.VMEM((CHUNK, 16), jnp.float32),
            pltpu.VMEM((ROWS, WIDE), jnp.float32),
            pltpu.VMEM((V_LAST, D), jnp.float32),
            pltpu.SemaphoreType.DMA,
            pltpu.SemaphoreType.DMA,
        ),
        compiler_params=pltpu.CompilerParams(use_tc_tiling_on_sc=False,
                                             needs_layout_passes=False),
    )
    def k(x_hbm, emb_hbm, lin_hbm, off_hbm, tail_hbm, out_hbm,
          idxb, lidxb, offb, ebuf, lbuf, obuf, tailbuf, sem_e, sem_l):
        wid = lax.axis_index("s") * NC + lax.axis_index("c")
        base = pl.multiple_of(wid * PER_W, 8)
        row_base = pl.multiple_of(wid * (PER_W // F), 8)
        pltpu.sync_copy(off_hbm, offb)
        pltpu.sync_copy(tail_hbm, tailbuf)
        lane_iota = lax.iota(jnp.int32, 16)
        zeros16 = jnp.zeros((16,), jnp.float32)

        def zinit(r, c):
            for col in range(D_IN, WIDE, 16):
                obuf[r, pl.ds(col, 16)] = zeros16
            return c

        lax.fori_loop(0, ROWS, zinit, 0)

        def chunk(j, carry):
            s0 = pl.multiple_of(base + j * CHUNK, 8)
            pltpu.sync_copy(x_hbm.at[pl.ds(s0, CHUNK)], idxb)

            def add(i, c):
                s = pl.ds(pl.multiple_of(i * 16, 16), 16)
                xi = idxb[s] + offb[s]
                idxb[s] = xi
                lidxb[s] = lax.shift_right_logical(xi, 4)
                return c

            lax.fori_loop(0, CHUNK // 16, add, 0)
            ce = pltpu.async_copy(emb_hbm.at[idxb], ebuf, sem_e)
            cl = pltpu.async_copy(lin_hbm.at[lidxb], lbuf, sem_l)
            ce.wait()
            cl.wait()

            def sel(i, c):
                s = pl.ds(pl.multiple_of(i * 16, 16), 16)
                col = lax.bitwise_and(idxb[s], 15)
                row = lane_iota + i * 16
                vals = plsc.load_gather(lbuf, [row, col])
                p = lane_iota + i * 16
                orow = lax.div(p, jnp.int32(F))
                ocol = D_IN + lax.rem(p, jnp.int32(F))
                plsc.store_scatter(obuf, [orow, ocol], vals)
                return c

            lax.fori_loop(0, CHUNK // 16, sel, 0)

            def repack(r, c):
                for kf in range(F):
                    obuf[r, pl.ds(kf * 16, 16)] = ebuf[r * F + kf, :]
                return c

            lax.fori_loop(0, ROWS, repack, 0)

            def fixup(i, c):
                s = pl.ds(pl.multiple_of(i * 16, 16), 16)
                xi = idxb[s]
                m = xi >= V_TAIL
                cnt = jnp.max(m.astype(jnp.int32))

                @pl.when(cnt > 0)
                def _():
                    p = lane_iota + i * 16
                    orow = lax.div(p, jnp.int32(F))
                    ocol0 = lax.rem(p, jnp.int32(F)) * 16
                    t = jnp.clip(xi - V_TAIL, 0, V_LAST - 1)
                    for d in range(D):
                        dv = jnp.full((16,), d, jnp.int32)
                        vals = plsc.load_gather(tailbuf, [t, dv], mask=m)
                        plsc.store_scatter(obuf, [orow, ocol0 + d], vals,
                                           mask=m)
                return c

            lax.fori_loop(0, CHUNK // 16, fixup, 0)
            r0 = pl.multiple_of(row_base + j * ROWS, 8)
            pltpu.sync_copy(obuf, out_hbm.at[pl.ds(r0, ROWS)])
            return carry

        lax.fori_loop(0, CHUNKS, chunk, 0)

    return k(x_flat, emb_table, lin16, off_flat, tail16)


def _tc_body(h_ref, w1_ref, b1_ref, w2_ref, b2_ref, w3_ref, b3_ref,
             s_ref, m_ref, out_ref):
    h = h_ref[...]
    se = jnp.dot(h, s_ref[...], preferred_element_type=jnp.float32)
    msel = jnp.dot(h * h, m_ref[...], preferred_element_type=jnp.float32)
    ysel = jnp.dot(h, m_ref[...], preferred_element_type=jnp.float32)
    sum_sq = msel[:, 0:1]
    ylin = ysel[:, 1:2]
    inter = 0.5 * (jnp.sum(se * se, axis=1, keepdims=True) - sum_sq)
    a = jnp.dot(h, w1_ref[...], preferred_element_type=jnp.float32) + b1_ref[...]
    a = jnp.maximum(a, 0.0)
    a = jnp.dot(a, w2_ref[...], preferred_element_type=jnp.float32) + b2_ref[...]
    a = jnp.maximum(a, 0.0)
    yd = jnp.dot(a, w3_ref[...], preferred_element_type=jnp.float32)
    out_ref[...] = yd + inter + ylin + b3_ref[...]


def _tc_mlp(h, W1p, b1, W2, b2, W3, b3c, Sp, Mp):
    grid = (B // BB,)
    return pl.pallas_call(
        _tc_body,
        grid=grid,
        in_specs=[
            pl.BlockSpec((BB, WIDE), lambda i: (i, 0)),
            pl.BlockSpec((WIDE, H1), lambda i: (0, 0)),
            pl.BlockSpec((1, H1), lambda i: (0, 0)),
            pl.BlockSpec((H1, H2), lambda i: (0, 0)),
            pl.BlockSpec((1, H2), lambda i: (0, 0)),
            pl.BlockSpec((H2, 1), lambda i: (0, 0)),
            pl.BlockSpec((1, 1), lambda i: (0, 0)),
            pl.BlockSpec((WIDE, D), lambda i: (0, 0)),
            pl.BlockSpec((WIDE, 2), lambda i: (0, 0)),
        ],
        out_specs=pl.BlockSpec((BB, 1), lambda i: (i, 0)),
        out_shape=jax.ShapeDtypeStruct((B, 1), jnp.float32),
    )(h, W1p, b1, W2, b2, W3, b3c, Sp, Mp)


def kernel(x, emb_table, lin_table, lin_bias, W1, b1, W2, b2, W3, b3):
    x_flat = x.reshape(N)
    pos = np.arange(CHUNK, dtype=np.int64)
    off_flat = jnp.asarray(((pos % F) * VOCAB_PER_FIELD).astype(np.int32))
    emb_wide = _sc_untile(emb_table.T)
    emb16 = emb_wide.reshape(-1, D)
    lin16 = lin_table.T.reshape(-1, 16)
    tail16 = emb_table[V_TAIL:, :]
    h = _sc_gather(x_flat, emb16, lin16, off_flat, tail16)
    W1p = jnp.concatenate([W1, jnp.zeros((WIDE - D_IN, H1), jnp.float32)], axis=0)
    s_np = np.zeros((WIDE, D), np.float32)
    s_np[:D_IN] = np.tile(np.eye(D, dtype=np.float32), (F, 1))
    m_np = np.zeros((WIDE, 2), np.float32)
    m_np[:D_IN, 0] = 1.0
    m_np[D_IN:D_IN + F, 1] = 1.0
    y = _tc_mlp(h, W1p, b1.reshape(1, H1), W2, b2.reshape(1, H2), W3,
                (b3 + lin_bias).reshape(1, 1), jnp.asarray(s_np),
                jnp.asarray(m_np))
    return y.reshape(B)


# speedup vs baseline: 1.1635x; 1.1635x over previous
import functools

import jax
import jax.numpy as jnp
import numpy as np
from jax import lax
from jax.experimental import pallas as pl
from jax.experimental.pallas import tpu as pltpu
from jax.experimental.pallas import tpu_sc as plsc

B = 16384
F = 26
D = 16
VOCAB_PER_FIELD = 100000
N = B * F
NC, NS = 2, 16
NW = NC * NS
PER_W = N // NW
CHUNK = 1664
ROWS = CHUNK // F
CHUNKS = PER_W // CHUNK
D_IN = F * D
WIDE = 512
H1, H2 = 256, 128
BB = 1024


V = 2600000
NGRP = V // 8
VC = 1024
VCHUNKS = -(-V // VC)
W_CHUNKS = -(-VCHUNKS // NW)
V_TAIL = (V // 128) * 128
V_LAST = V - V_TAIL


def _sc_untile(embT):
    mesh = plsc.VectorSubcoreMesh(core_axis_name="c", subcore_axis_name="s")

    @functools.partial(
        pl.kernel,
        out_type=jax.ShapeDtypeStruct((NGRP, 128), jnp.float32),
        mesh=mesh,
        scratch_types=(
            pltpu.VMEM((D, VC), jnp.float32),
            pltpu.VMEM((D, VC), jnp.float32),
            pltpu.VMEM((VC // 8, 128), jnp.float32),
            pltpu.VMEM((VC // 8, 128), jnp.float32),
            pltpu.SemaphoreType.DMA,
            pltpu.SemaphoreType.DMA,
        ),
        compiler_params=pltpu.CompilerParams(use_tc_tiling_on_sc=True,
                                             needs_layout_passes=False),
    )
    def k(embT_hbm, out_hbm, tbuf_a, tbuf_b, obuf_a, obuf_b, sem_a, sem_b):
        wid = lax.axis_index("s") * NC + lax.axis_index("c")
        lane_iota = lax.iota(jnp.int32, 16)

        def v_of(t):
            c = wid + t * NW
            return pl.multiple_of(jnp.minimum(c * VC, V_TAIL - VC), 128)

        def fire(t, tbuf, sem):
            return pltpu.async_copy(embT_hbm.at[:, pl.ds(v_of(t), VC)],
                                    tbuf, sem)

        def process(t, tbuf, obuf):
            def tr(i, cc):
                vv = lane_iota + i * 16
                orow = lax.shift_right_logical(vv, 3)
                ocol0 = lax.bitwise_and(vv, 7) * 16
                for d in range(D):
                    vals = tbuf[d, pl.ds(pl.multiple_of(i * 16, 16), 16)]
                    plsc.store_scatter(obuf, [orow, ocol0 + d], vals)
                return cc

            lax.fori_loop(0, VC // 16, tr, 0)
            r0 = pl.multiple_of(lax.shift_right_logical(v_of(t), 3), 8)
            pltpu.sync_copy(obuf, out_hbm.at[pl.ds(r0, VC // 8)])

        fire(0, tbuf_a, sem_a)

        def pair(u, carry):
            t = u * 2
            cb = fire(t + 1, tbuf_b, sem_b)
            pltpu.make_async_copy(embT_hbm.at[:, pl.ds(v_of(t), VC)],
                                  tbuf_a, sem_a).wait()
            process(t, tbuf_a, obuf_a)
            fire(t + 2, tbuf_a, sem_a)
            cb.wait()
            process(t + 1, tbuf_b, obuf_b)
            return carry

        lax.fori_loop(0, W_CHUNKS // 2, pair, 0)
        pltpu.make_async_copy(embT_hbm.at[:, pl.ds(v_of(W_CHUNKS), VC)],
                              tbuf_a, sem_a).wait()


    return k(embT)


def _sc_gather(x_flat, emb_table, lin16, off_flat, tail16):
    mesh = plsc.VectorSubcoreMesh(core_axis_name="c", subcore_axis_name="s")

    @functools.partial(
        pl.kernel,
        out_type=jax.ShapeDtypeStruct((B, WIDE), jnp.float32),
        mesh=mesh,
        scratch_types=(
            pltpu.VMEM((CHUNK,), jnp.int32),
            pltpu.VMEM((CHUNK,), jnp.int32),
            pltpu.VMEM((CHUNK,), jnp.int32),
            pltpu.VMEM((CHUNK, D), jnp.float32),
            pltpu.VMEM((CHUNK, 16), jnp.float32),
            pltpu.VMEM((ROWS, WIDE), jnp.float32),
            pltpu.VMEM((V_LAST, D), jnp.float32),
            pltpu.SemaphoreType.DMA,
            pltpu.SemaphoreType.DMA,
        ),
        compiler_params=pltpu.CompilerParams(use_tc_tiling_on_sc=False,
                                             needs_layout_passes=False),
    )
    def k(x_hbm, emb_hbm, lin_hbm, off_hbm, tail_hbm, out_hbm,
          idxb, lidxb, offb, ebuf, lbuf, obuf, tailbuf, sem_e, sem_l):
        wid = lax.axis_index("s") * NC + lax.axis_index("c")
        base = pl.multiple_of(wid * PER_W, 8)
        row_base = pl.multiple_of(wid * (PER_W // F), 8)
        pltpu.sync_copy(off_hbm, offb)
        pltpu.sync_copy(tail_hbm, tailbuf)
        lane_iota = lax.iota(jnp.int32, 16)
        zeros16 = jnp.zeros((16,), jnp.float32)

        def zinit(r, c):
            for col in range(D_IN, WIDE, 16):
                obuf[r, pl.ds(col, 16)] = zeros16
            return c

        lax.fori_loop(0, ROWS, zinit, 0)

        def chunk(j, carry):
            s0 = pl.multiple_of(base + j * CHUNK, 8)
            pltpu.sync_copy(x_hbm.at[pl.ds(s0, CHUNK)], idxb)

            def add(i, c):
                s = pl.ds(pl.multiple_of(i * 16, 16), 16)
                xi = idxb[s] + offb[s]
                idxb[s] = xi
                lidxb[s] = lax.shift_right_logical(xi, 4)
                return c

            lax.fori_loop(0, CHUNK // 16, add, 0)
            ce = pltpu.async_copy(emb_hbm.at[idxb], ebuf, sem_e)
            cl = pltpu.async_copy(lin_hbm.at[lidxb], lbuf, sem_l)
            ce.wait()
            cl.wait()

            def sel(i, c):
                s = pl.ds(pl.multiple_of(i * 16, 16), 16)
                col = lax.bitwise_and(idxb[s], 15)
                row = lane_iota + i * 16
                vals = plsc.load_gather(lbuf, [row, col])
                p = lane_iota + i * 16
                orow = lax.div(p, jnp.int32(F))
                ocol = D_IN + lax.rem(p, jnp.int32(F))
                plsc.store_scatter(obuf, [orow, ocol], vals)
                return c

            lax.fori_loop(0, CHUNK // 16, sel, 0)

            def repack(r, c):
                for kf in range(F):
                    obuf[r, pl.ds(kf * 16, 16)] = ebuf[r * F + kf, :]
                return c

            lax.fori_loop(0, ROWS, repack, 0)

            def fixup(i, c):
                s = pl.ds(pl.multiple_of(i * 16, 16), 16)
                xi = idxb[s]
                m = xi >= V_TAIL
                cnt = jnp.max(m.astype(jnp.int32))

                @pl.when(cnt > 0)
                def _():
                    p = lane_iota + i * 16
                    orow = lax.div(p, jnp.int32(F))
                    ocol0 = lax.rem(p, jnp.int32(F)) * 16
                    t = jnp.clip(xi - V_TAIL, 0, V_LAST - 1)
                    for d in range(D):
                        dv = jnp.full((16,), d, jnp.int32)
                        vals = plsc.load_gather(tailbuf, [t, dv], mask=m)
                        plsc.store_scatter(obuf, [orow, ocol0 + d], vals,
                                           mask=m)
                return c

            lax.fori_loop(0, CHUNK // 16, fixup, 0)
            r0 = pl.multiple_of(row_base + j * ROWS, 8)
            pltpu.sync_copy(obuf, out_hbm.at[pl.ds(r0, ROWS)])
            return carry

        lax.fori_loop(0, CHUNKS, chunk, 0)

    return k(x_flat, emb_table, lin16, off_flat, tail16)


def _tc_body(h_ref, w1_ref, b1_ref, w2_ref, b2_ref, w3_ref, b3_ref,
             s_ref, m_ref, out_ref):
    h = h_ref[...]
    se = jnp.dot(h, s_ref[...], preferred_element_type=jnp.float32)
    msel = jnp.dot(h * h, m_ref[...], preferred_element_type=jnp.float32)
    ysel = jnp.dot(h, m_ref[...], preferred_element_type=jnp.float32)
    sum_sq = msel[:, 0:1]
    ylin = ysel[:, 1:2]
    inter = 0.5 * (jnp.sum(se * se, axis=1, keepdims=True) - sum_sq)
    a = jnp.dot(h, w1_ref[...], preferred_element_type=jnp.float32) + b1_ref[...]
    a = jnp.maximum(a, 0.0)
    a = jnp.dot(a, w2_ref[...], preferred_element_type=jnp.float32) + b2_ref[...]
    a = jnp.maximum(a, 0.0)
    yd = jnp.dot(a, w3_ref[...], preferred_element_type=jnp.float32)
    out_ref[...] = yd + inter + ylin + b3_ref[...]


def _tc_mlp(h, W1p, b1, W2, b2, W3, b3c, Sp, Mp):
    grid = (B // BB,)
    return pl.pallas_call(
        _tc_body,
        grid=grid,
        in_specs=[
            pl.BlockSpec((BB, WIDE), lambda i: (i, 0)),
            pl.BlockSpec((WIDE, H1), lambda i: (0, 0)),
            pl.BlockSpec((1, H1), lambda i: (0, 0)),
            pl.BlockSpec((H1, H2), lambda i: (0, 0)),
            pl.BlockSpec((1, H2), lambda i: (0, 0)),
            pl.BlockSpec((H2, 1), lambda i: (0, 0)),
            pl.BlockSpec((1, 1), lambda i: (0, 0)),
            pl.BlockSpec((WIDE, D), lambda i: (0, 0)),
            pl.BlockSpec((WIDE, 2), lambda i: (0, 0)),
        ],
        out_specs=pl.BlockSpec((BB, 1), lambda i: (i, 0)),
        out_shape=jax.ShapeDtypeStruct((B, 1), jnp.float32),
    )(h, W1p, b1, W2, b2, W3, b3c, Sp, Mp)


def kernel(x, emb_table, lin_table, lin_bias, W1, b1, W2, b2, W3, b3):
    x_flat = x.reshape(N)
    pos = np.arange(CHUNK, dtype=np.int64)
    off_flat = jnp.asarray(((pos % F) * VOCAB_PER_FIELD).astype(np.int32))
    emb_wide = _sc_untile(emb_table.T)
    emb16 = emb_wide.reshape(-1, D)
    lin16 = lin_table.T.reshape(-1, 16)
    tail16 = emb_table[V_TAIL:, :]
    h = _sc_gather(x_flat, emb16, lin16, off_flat, tail16)
    W1p = jnp.concatenate([W1, jnp.zeros((WIDE - D_IN, H1), jnp.float32)], axis=0)
    s_np = np.zeros((WIDE, D), np.float32)
    s_np[:D_IN] = np.tile(np.eye(D, dtype=np.float32), (F, 1))
    m_np = np.zeros((WIDE, 2), np.float32)
    m_np[:D_IN, 0] = 1.0
    m_np[D_IN:D_IN + F, 1] = 1.0
    y = _tc_mlp(h, W1p, b1.reshape(1, H1), W2, b2.reshape(1, H2), W3,
                (b3 + lin_bias).reshape(1, 1), jnp.asarray(s_np),
                jnp.asarray(m_np))
    return y.reshape(B)


# speedup vs baseline: 1.1980x; 1.0297x over previous
import functools

import jax
import jax.numpy as jnp
import numpy as np
from jax import lax
from jax.experimental import pallas as pl
from jax.experimental.pallas import tpu as pltpu
from jax.experimental.pallas import tpu_sc as plsc

B = 16384
F = 26
D = 16
VOCAB_PER_FIELD = 100000
N = B * F
NC, NS = 2, 16
NW = NC * NS
PER_W = N // NW
CHUNK = 832
ROWS = CHUNK // F
CHUNKS = PER_W // CHUNK
D_IN = F * D
WIDE = 512
H1, H2 = 256, 128
BB = 1024


V = 2600000
NGRP = V // 8
VC = 1024
VCHUNKS = -(-V // VC)
W_CHUNKS = -(-VCHUNKS // NW)
V_TAIL = (V // 128) * 128
V_LAST = V - V_TAIL


def _sc_untile(embT):
    mesh = plsc.VectorSubcoreMesh(core_axis_name="c", subcore_axis_name="s")

    @functools.partial(
        pl.kernel,
        out_type=jax.ShapeDtypeStruct((NGRP, 128), jnp.float32),
        mesh=mesh,
        scratch_types=(
            pltpu.VMEM((D, VC), jnp.float32),
            pltpu.VMEM((D, VC), jnp.float32),
            pltpu.VMEM((VC // 8, 128), jnp.float32),
            pltpu.VMEM((VC // 8, 128), jnp.float32),
            pltpu.SemaphoreType.DMA,
            pltpu.SemaphoreType.DMA,
        ),
        compiler_params=pltpu.CompilerParams(use_tc_tiling_on_sc=True,
                                             needs_layout_passes=False),
    )
    def k(embT_hbm, out_hbm, tbuf_a, tbuf_b, obuf_a, obuf_b, sem_a, sem_b):
        wid = lax.axis_index("s") * NC + lax.axis_index("c")
        lane_iota = lax.iota(jnp.int32, 16)

        def v_of(t):
            c = wid + t * NW
            return pl.multiple_of(jnp.minimum(c * VC, V_TAIL - VC), 128)

        def fire(t, tbuf, sem):
            return pltpu.async_copy(embT_hbm.at[:, pl.ds(v_of(t), VC)],
                                    tbuf, sem)

        def process(t, tbuf, obuf):
            def tr(i, cc):
                vv = lane_iota + i * 16
                orow = lax.shift_right_logical(vv, 3)
                ocol0 = lax.bitwise_and(vv, 7) * 16
                for d in range(D):
                    vals = tbuf[d, pl.ds(pl.multiple_of(i * 16, 16), 16)]
                    plsc.store_scatter(obuf, [orow, ocol0 + d], vals)
                return cc

            lax.fori_loop(0, VC // 16, tr, 0)
            r0 = pl.multiple_of(lax.shift_right_logical(v_of(t), 3), 8)
            pltpu.sync_copy(obuf, out_hbm.at[pl.ds(r0, VC // 8)])

        fire(0, tbuf_a, sem_a)

        def pair(u, carry):
            t = u * 2
            cb = fire(t + 1, tbuf_b, sem_b)
            pltpu.make_async_copy(embT_hbm.at[:, pl.ds(v_of(t), VC)],
                                  tbuf_a, sem_a).wait()
            process(t, tbuf_a, obuf_a)
            fire(t + 2, tbuf_a, sem_a)
            cb.wait()
            process(t + 1, tbuf_b, obuf_b)
            return carry

        lax.fori_loop(0, W_CHUNKS // 2, pair, 0)
        pltpu.make_async_copy(embT_hbm.at[:, pl.ds(v_of(W_CHUNKS), VC)],
                              tbuf_a, sem_a).wait()


    return k(embT)


def _sc_gather(x_flat, emb_table, lin16, off_flat, tail16):
    mesh = plsc.VectorSubcoreMesh(core_axis_name="c", subcore_axis_name="s")

    @functools.partial(
        pl.kernel,
        out_type=jax.ShapeDtypeStruct((B, WIDE), jnp.float32),
        mesh=mesh,
        scratch_types=(
            pltpu.VMEM((CHUNK,), jnp.int32),
            pltpu.VMEM((CHUNK,), jnp.int32),
            pltpu.VMEM((CHUNK,), jnp.int32),
            pltpu.VMEM((CHUNK,), jnp.int32),
            pltpu.VMEM((CHUNK,), jnp.int32),
            pltpu.VMEM((CHUNK, D), jnp.float32),
            pltpu.VMEM((CHUNK, D), jnp.float32),
            pltpu.VMEM((CHUNK, 16), jnp.float32),
            pltpu.VMEM((CHUNK, 16), jnp.float32),
            pltpu.VMEM((ROWS, WIDE), jnp.float32),
            pltpu.VMEM((V_LAST, D), jnp.float32),
            pltpu.SemaphoreType.DMA,
            pltpu.SemaphoreType.DMA,
            pltpu.SemaphoreType.DMA,
            pltpu.SemaphoreType.DMA,
        ),
        compiler_params=pltpu.CompilerParams(use_tc_tiling_on_sc=False,
                                             needs_layout_passes=False),
    )
    def k(x_hbm, emb_hbm, lin_hbm, off_hbm, tail_hbm, out_hbm,
          idx_a, idx_b, lidx_a, lidx_b, offb, ebuf_a, ebuf_b, lbuf_a, lbuf_b,
          obuf, tailbuf, sem_ea, sem_eb, sem_la, sem_lb):
        wid = lax.axis_index("s") * NC + lax.axis_index("c")
        base = pl.multiple_of(wid * PER_W, 8)
        row_base = pl.multiple_of(wid * (PER_W // F), 8)
        pltpu.sync_copy(off_hbm, offb)
        pltpu.sync_copy(tail_hbm, tailbuf)
        lane_iota = lax.iota(jnp.int32, 16)
        zeros16 = jnp.zeros((16,), jnp.float32)

        def zinit(r, c):
            for col in range(D_IN, WIDE, 16):
                obuf[r, pl.ds(col, 16)] = zeros16
            return c

        lax.fori_loop(0, ROWS, zinit, 0)

        def s_of(j):
            return pl.multiple_of(
                jnp.minimum(base + j * CHUNK, N - CHUNK), 8)

        def prep(j, idxb, lidxb, ebuf, lbuf, sem_e, sem_l):
            pltpu.sync_copy(x_hbm.at[pl.ds(s_of(j), CHUNK)], idxb)

            def add(i, c):
                s = pl.ds(pl.multiple_of(i * 16, 16), 16)
                xi = idxb[s] + offb[s]
                idxb[s] = xi
                lidxb[s] = lax.shift_right_logical(xi, 4)
                return c

            lax.fori_loop(0, CHUNK // 16, add, 0)
            pltpu.async_copy(emb_hbm.at[idxb], ebuf, sem_e)
            pltpu.async_copy(lin_hbm.at[lidxb], lbuf, sem_l)

        def drain(idxb, lidxb, ebuf, lbuf, sem_e, sem_l):
            pltpu.make_async_copy(emb_hbm.at[idxb], ebuf, sem_e).wait()
            pltpu.make_async_copy(lin_hbm.at[lidxb], lbuf, sem_l).wait()

        def process(j, idxb, ebuf, lbuf):
            def sel(i, c):
                s = pl.ds(pl.multiple_of(i * 16, 16), 16)
                col = lax.bitwise_and(idxb[s], 15)
                row = lane_iota + i * 16
                vals = plsc.load_gather(lbuf, [row, col])
                p = lane_iota + i * 16
                orow = lax.div(p, jnp.int32(F))
                ocol = D_IN + lax.rem(p, jnp.int32(F))
                plsc.store_scatter(obuf, [orow, ocol], vals)
                return c

            lax.fori_loop(0, CHUNK // 16, sel, 0)

            def repack(r, c):
                for kf in range(F):
                    obuf[r, pl.ds(kf * 16, 16)] = ebuf[r * F + kf, :]
                return c

            lax.fori_loop(0, ROWS, repack, 0)

            def fixup(i, c):
                s = pl.ds(pl.multiple_of(i * 16, 16), 16)
                xi = idxb[s]
                m = xi >= V_TAIL
                cnt = jnp.max(m.astype(jnp.int32))

                @pl.when(cnt > 0)
                def _():
                    p = lane_iota + i * 16
                    orow = lax.div(p, jnp.int32(F))
                    ocol0 = lax.rem(p, jnp.int32(F)) * 16
                    t = jnp.clip(xi - V_TAIL, 0, V_LAST - 1)
                    for d in range(D):
                        dv = jnp.full((16,), d, jnp.int32)
                        vals = plsc.load_gather(tailbuf, [t, dv], mask=m)
                        plsc.store_scatter(obuf, [orow, ocol0 + d], vals,
                                           mask=m)
                return c

            lax.fori_loop(0, CHUNK // 16, fixup, 0)
            r0 = pl.multiple_of(row_base + j * ROWS, 8)
            pltpu.sync_copy(obuf, out_hbm.at[pl.ds(r0, ROWS)])

        prep(0, idx_a, lidx_a, ebuf_a, lbuf_a, sem_ea, sem_la)

        def pair(u, carry):
            j = u * 2
            prep(j + 1, idx_b, lidx_b, ebuf_b, lbuf_b, sem_eb, sem_lb)
            drain(idx_a, lidx_a, ebuf_a, lbuf_a, sem_ea, sem_la)
            process(j, idx_a, ebuf_a, lbuf_a)
            prep(j + 2, idx_a, lidx_a, ebuf_a, lbuf_a, sem_ea, sem_la)
            drain(idx_b, lidx_b, ebuf_b, lbuf_b, sem_eb, sem_lb)
            process(j + 1, idx_b, ebuf_b, lbuf_b)
            return carry

        lax.fori_loop(0, CHUNKS // 2, pair, 0)
        drain(idx_a, lidx_a, ebuf_a, lbuf_a, sem_ea, sem_la)

    return k(x_flat, emb_table, lin16, off_flat, tail16)


def _tc_body(h_ref, w1_ref, b1_ref, w2_ref, b2_ref, w3_ref, b3_ref,
             s_ref, m_ref, out_ref):
    h = h_ref[...]
    se = jnp.dot(h, s_ref[...], preferred_element_type=jnp.float32)
    msel = jnp.dot(h * h, m_ref[...], preferred_element_type=jnp.float32)
    ysel = jnp.dot(h, m_ref[...], preferred_element_type=jnp.float32)
    sum_sq = msel[:, 0:1]
    ylin = ysel[:, 1:2]
    inter = 0.5 * (jnp.sum(se * se, axis=1, keepdims=True) - sum_sq)
    a = jnp.dot(h, w1_ref[...], preferred_element_type=jnp.float32) + b1_ref[...]
    a = jnp.maximum(a, 0.0)
    a = jnp.dot(a, w2_ref[...], preferred_element_type=jnp.float32) + b2_ref[...]
    a = jnp.maximum(a, 0.0)
    yd = jnp.dot(a, w3_ref[...], preferred_element_type=jnp.float32)
    out_ref[...] = yd + inter + ylin + b3_ref[...]


def _tc_mlp(h, W1p, b1, W2, b2, W3, b3c, Sp, Mp):
    grid = (B // BB,)
    return pl.pallas_call(
        _tc_body,
        grid=grid,
        in_specs=[
            pl.BlockSpec((BB, WIDE), lambda i: (i, 0)),
            pl.BlockSpec((WIDE, H1), lambda i: (0, 0)),
            pl.BlockSpec((1, H1), lambda i: (0, 0)),
            pl.BlockSpec((H1, H2), lambda i: (0, 0)),
            pl.BlockSpec((1, H2), lambda i: (0, 0)),
            pl.BlockSpec((H2, 1), lambda i: (0, 0)),
            pl.BlockSpec((1, 1), lambda i: (0, 0)),
            pl.BlockSpec((WIDE, D), lambda i: (0, 0)),
            pl.BlockSpec((WIDE, 2), lambda i: (0, 0)),
        ],
        out_specs=pl.BlockSpec((BB, 1), lambda i: (i, 0)),
        out_shape=jax.ShapeDtypeStruct((B, 1), jnp.float32),
    )(h, W1p, b1, W2, b2, W3, b3c, Sp, Mp)


def kernel(x, emb_table, lin_table, lin_bias, W1, b1, W2, b2, W3, b3):
    x_flat = x.reshape(N)
    pos = np.arange(CHUNK, dtype=np.int64)
    off_flat = jnp.asarray(((pos % F) * VOCAB_PER_FIELD).astype(np.int32))
    emb_wide = _sc_untile(emb_table.T)
    emb16 = emb_wide.reshape(-1, D)
    lin16 = lin_table.T.reshape(-1, 16)
    tail16 = emb_table[V_TAIL:, :]
    h = _sc_gather(x_flat, emb16, lin16, off_flat, tail16)
    W1p = jnp.concatenate([W1, jnp.zeros((WIDE - D_IN, H1), jnp.float32)], axis=0)
    s_np = np.zeros((WIDE, D), np.float32)
    s_np[:D_IN] = np.tile(np.eye(D, dtype=np.float32), (F, 1))
    m_np = np.zeros((WIDE, 2), np.float32)
    m_np[:D_IN, 0] = 1.0
    m_np[D_IN:D_IN + F, 1] = 1.0
    y = _tc_mlp(h, W1p, b1.reshape(1, H1), W2, b2.reshape(1, H2), W3,
                (b3 + lin_bias).reshape(1, 1), jnp.asarray(s_np),
                jnp.asarray(m_np))
    return y.reshape(B)
